# Initial kernel scaffold; baseline (speedup 1.0000x reference)
import jax
import jax.numpy as jnp
from jax.experimental import pallas as pl


def kernel(x, wb_coarse, wb_mid, wb_fine, Wq, bq, Wk, bk, Wv, bv, Wo, bo):
    raise NotImplementedError("write your pallas kernel here")


# speedup vs baseline: 8.4380x; 8.4380x over previous
import math

import jax
import jax.numpy as jnp
from jax.experimental import pallas as pl

_CSIZES = (512, 128, 32)
_NUM_HEADS = 16


def _rank_kernel(wbc_ref, wbr_ref, pos_ref):
    bj = wbr_ref[0, 0]
    n = bj.shape[1]
    tr = 256
    for t in range(n // tr):
        bi = wbc_ref[0, 0, t * tr:(t + 1) * tr, :]
        lt = (bj < bi)
        col = jax.lax.broadcasted_iota(jnp.int32, (tr, n), 1)
        row = jax.lax.broadcasted_iota(jnp.int32, (tr, n), 0) + (t * tr)
        eqb = (bj == bi) & (col < row)
        cnt = jnp.sum(lt.astype(jnp.int32) + eqb.astype(jnp.int32), axis=1,
                      keepdims=True)
        pos_ref[0, 0, t * tr:(t + 1) * tr, :] = cnt


def _qkv_kernel(x_ref, wq_ref, wk_ref, wv_ref, bq_ref, bk_ref, bv_ref,
                q_ref, k_ref, v_ref):
    xb = x_ref[0]
    dn = (((1,), (1,)), ((), ()))
    q_ref[0, 0] = (jax.lax.dot_general(xb, wq_ref[...], dn,
                                       preferred_element_type=jnp.float32)
                   + bq_ref[0])
    k_ref[0, 0] = (jax.lax.dot_general(xb, wk_ref[...], dn,
                                       preferred_element_type=jnp.float32)
                   + bk_ref[0])
    v_ref[0, 0] = (jax.lax.dot_general(xb, wv_ref[...], dn,
                                       preferred_element_type=jnp.float32)
                   + bv_ref[0])


def _attn_kernel(q_ref, k_ref, v_ref, posq_ref, posk_ref, o_ref):
    q = q_ref[0, 0]
    k = k_ref[0, 0]
    v = v_ref[0, 0]
    tq, d = q.shape
    n = k.shape[0]
    s = jax.lax.dot_general(q, k, (((1,), (1,)), ((), ())),
                            preferred_element_type=jnp.float32)
    s = s * (1.0 / math.sqrt(d))
    neg = jnp.float32(-1e30)
    m = jnp.full((tq, 1), neg, jnp.float32)
    masks = []
    c0s = []
    for lvl, cs in enumerate(_CSIZES):
        shift = cs.bit_length() - 1
        cq = posq_ref[lvl, 0] >> shift
        ck = posk_ref[lvl, 0] >> shift
        diff = cq - ck
        mask = (diff == 0) | (diff == 1)
        c0 = (cq == 0)
        m_l = jnp.max(jnp.where(mask, s, neg), axis=1, keepdims=True)
        m_l = jnp.where(c0, jnp.maximum(m_l, 0.0), m_l)
        m = jnp.maximum(m, m_l)
        masks.append(mask)
        c0s.append(c0)
    e = jnp.exp(s - m)
    expm = jnp.exp(-m)
    p = jnp.zeros((tq, n), jnp.float32)
    for lvl, cs in enumerate(_CSIZES):
        num = jnp.where(masks[lvl], e, 0.0)
        denom = jnp.sum(num, axis=1, keepdims=True)
        denom = denom + jnp.where(c0s[lvl], jnp.float32(cs) * expm, 0.0)
        denom = jnp.maximum(denom, jnp.float32(1e-37))
        p = p + num / denom
    o = jax.lax.dot_general(p, v, (((1,), (0,)), ((), ())),
                            preferred_element_type=jnp.float32)
    o_ref[0, 0] = o * (1.0 / 3.0)


def _oproj_kernel(a_ref, wo_ref, bo_ref, o_ref):
    h = pl.program_id(1)
    a = a_ref[0, 0]
    acc = jax.lax.dot_general(a, wo_ref[...], (((1,), (1,)), ((), ())),
                              preferred_element_type=jnp.float32)

    @pl.when(h == 0)
    def _():
        o_ref[0] = acc + bo_ref[0]

    @pl.when(h != 0)
    def _():
        o_ref[0] = o_ref[0] + acc


def kernel(x, wb_coarse, wb_mid, wb_fine, Wq, bq, Wk, bk, Wv, bv, Wo, bo):
    B, N, E = x.shape
    H = _NUM_HEADS
    D = E // H
    TQ = 512
    f32 = jnp.float32

    wb_all = jnp.stack([wb_coarse, wb_mid, wb_fine]).astype(jnp.int32)
    wbc = wb_all[:, :, :, None]
    wbr = wb_all[:, :, None, :]
    posc = pl.pallas_call(
        _rank_kernel,
        grid=(3, B),
        in_specs=[
            pl.BlockSpec((1, 1, N, 1), lambda l, b: (l, b, 0, 0)),
            pl.BlockSpec((1, 1, 1, N), lambda l, b: (l, b, 0, 0)),
        ],
        out_specs=pl.BlockSpec((1, 1, N, 1), lambda l, b: (l, b, 0, 0)),
        out_shape=jax.ShapeDtypeStruct((3, B, N, 1), jnp.int32),
    )(wbc, wbr)
    posr = jnp.swapaxes(posc, 2, 3)

    bq3 = bq.reshape(H, 1, D)
    bk3 = bk.reshape(H, 1, D)
    bv3 = bv.reshape(H, 1, D)
    qkv_shape = jax.ShapeDtypeStruct((B, H, N, D), f32)
    q, k, v = pl.pallas_call(
        _qkv_kernel,
        grid=(B, H),
        in_specs=[
            pl.BlockSpec((1, N, E), lambda b, h: (b, 0, 0)),
            pl.BlockSpec((D, E), lambda b, h: (h, 0)),
            pl.BlockSpec((D, E), lambda b, h: (h, 0)),
            pl.BlockSpec((D, E), lambda b, h: (h, 0)),
            pl.BlockSpec((1, 1, D), lambda b, h: (h, 0, 0)),
            pl.BlockSpec((1, 1, D), lambda b, h: (h, 0, 0)),
            pl.BlockSpec((1, 1, D), lambda b, h: (h, 0, 0)),
        ],
        out_specs=[pl.BlockSpec((1, 1, N, D), lambda b, h: (b, h, 0, 0))] * 3,
        out_shape=[qkv_shape] * 3,
    )(x, Wq, Wk, Wv, bq3, bk3, bv3)

    attn = pl.pallas_call(
        _attn_kernel,
        grid=(B, H, N // TQ),
        in_specs=[
            pl.BlockSpec((1, 1, TQ, D), lambda b, h, t: (b, h, t, 0)),
            pl.BlockSpec((1, 1, N, D), lambda b, h, t: (b, h, 0, 0)),
            pl.BlockSpec((1, 1, N, D), lambda b, h, t: (b, h, 0, 0)),
            pl.BlockSpec((3, 1, TQ, 1), lambda b, h, t: (0, b, t, 0)),
            pl.BlockSpec((3, 1, 1, N), lambda b, h, t: (0, b, 0, 0)),
        ],
        out_specs=pl.BlockSpec((1, 1, TQ, D), lambda b, h, t: (b, h, t, 0)),
        out_shape=jax.ShapeDtypeStruct((B, H, N, D), f32),
    )(q, k, v, posc, posr)

    out = pl.pallas_call(
        _oproj_kernel,
        grid=(B, H),
        in_specs=[
            pl.BlockSpec((1, 1, N, D), lambda b, h: (b, h, 0, 0)),
            pl.BlockSpec((E, D), lambda b, h: (0, h)),
            pl.BlockSpec((1, 1, E), lambda b, h: (0, 0, 0)),
        ],
        out_specs=pl.BlockSpec((1, N, E), lambda b, h: (b, 0, 0)),
        out_shape=jax.ShapeDtypeStruct((B, N, E), f32),
    )(attn, Wo, bo.reshape(1, 1, E))
    return out


# speedup vs baseline: 10.1242x; 1.1998x over previous
import math

import jax
import jax.numpy as jnp
from jax.experimental import pallas as pl

_CSIZES = (512, 128, 32)
_NUM_HEADS = 16


def _rank_kernel(wbc_ref, wbr_ref, pos_ref):
    bj = wbr_ref[0, 0]
    n = bj.shape[1]
    tr = 256
    for t in range(n // tr):
        bi = wbc_ref[0, 0, t * tr:(t + 1) * tr, :]
        lt = (bj < bi)
        col = jax.lax.broadcasted_iota(jnp.int32, (tr, n), 1)
        row = jax.lax.broadcasted_iota(jnp.int32, (tr, n), 0) + (t * tr)
        eqb = (bj == bi) & (col < row)
        cnt = jnp.sum(lt.astype(jnp.int32) + eqb.astype(jnp.int32), axis=1,
                      keepdims=True)
        pos_ref[0, 0, t * tr:(t + 1) * tr, :] = cnt


def _qkv_kernel(x_ref, wq_ref, wk_ref, wv_ref, bq_ref, bk_ref, bv_ref,
                q_ref, k_ref, v_ref):
    xb = x_ref[0]
    dn = (((1,), (1,)), ((), ()))
    q_ref[0, 0] = (jax.lax.dot_general(xb, wq_ref[...], dn,
                                       preferred_element_type=jnp.float32)
                   + bq_ref[0])
    k_ref[0, 0] = (jax.lax.dot_general(xb, wk_ref[...], dn,
                                       preferred_element_type=jnp.float32)
                   + bk_ref[0])
    v_ref[0, 0] = (jax.lax.dot_general(xb, wv_ref[...], dn,
                                       preferred_element_type=jnp.float32)
                   + bv_ref[0])


def _attn_kernel(q_ref, k_ref, v_ref, posq_ref, posk_ref, o_ref):
    q = q_ref[0, 0]
    k = k_ref[0, 0]
    v = v_ref[0, 0]
    tq, d = q.shape
    n = k.shape[0]
    s = jax.lax.dot_general(q, k, (((1,), (1,)), ((), ())),
                            preferred_element_type=jnp.float32)
    s = s * (1.0 / math.sqrt(d))
    m = jnp.maximum(jnp.max(s, axis=1, keepdims=True), 0.0)
    e = jnp.exp(s - m)
    expm = jnp.exp(-m)
    w = jnp.zeros((tq, n), jnp.float32)
    for lvl, cs in enumerate(_CSIZES):
        shift = cs.bit_length() - 1
        cq = posq_ref[lvl, 0] >> shift
        ck = posk_ref[lvl, 0] >> shift
        mask = (cq - ck).astype(jnp.uint32) <= jnp.uint32(1)
        den = jnp.sum(jnp.where(mask, e, 0.0), axis=1, keepdims=True)
        den = den + jnp.where(cq == 0, jnp.float32(cs) * expm, 0.0)
        r = 1.0 / jnp.maximum(den, jnp.float32(1e-37))
        w = w + jnp.where(mask, r, 0.0)
    p = e * w
    o = jax.lax.dot_general(p, v, (((1,), (0,)), ((), ())),
                            preferred_element_type=jnp.float32)
    o_ref[0] = o * (1.0 / 3.0)


def _oproj_kernel(a_ref, wo_ref, bo_ref, o_ref):
    o_ref[...] = (jax.lax.dot_general(a_ref[...], wo_ref[...],
                                      (((1,), (1,)), ((), ())),
                                      preferred_element_type=jnp.float32)
                  + bo_ref[...])


def kernel(x, wb_coarse, wb_mid, wb_fine, Wq, bq, Wk, bk, Wv, bv, Wo, bo):
    B, N, E = x.shape
    H = _NUM_HEADS
    D = E // H
    TQ = 512
    TM = 512
    f32 = jnp.float32

    wb_all = jnp.stack([wb_coarse, wb_mid, wb_fine]).astype(jnp.int32)
    wbc = wb_all[:, :, :, None]
    wbr = wb_all[:, :, None, :]
    posc = pl.pallas_call(
        _rank_kernel,
        grid=(3, B),
        in_specs=[
            pl.BlockSpec((1, 1, N, 1), lambda l, b: (l, b, 0, 0)),
            pl.BlockSpec((1, 1, 1, N), lambda l, b: (l, b, 0, 0)),
        ],
        out_specs=pl.BlockSpec((1, 1, N, 1), lambda l, b: (l, b, 0, 0)),
        out_shape=jax.ShapeDtypeStruct((3, B, N, 1), jnp.int32),
    )(wbc, wbr)
    posr = jnp.swapaxes(posc, 2, 3)

    bq3 = bq.reshape(H, 1, D)
    bk3 = bk.reshape(H, 1, D)
    bv3 = bv.reshape(H, 1, D)
    qkv_shape = jax.ShapeDtypeStruct((B, H, N, D), f32)
    q, k, v = pl.pallas_call(
        _qkv_kernel,
        grid=(B, H),
        in_specs=[
            pl.BlockSpec((1, N, E), lambda b, h: (b, 0, 0)),
            pl.BlockSpec((D, E), lambda b, h: (h, 0)),
            pl.BlockSpec((D, E), lambda b, h: (h, 0)),
            pl.BlockSpec((D, E), lambda b, h: (h, 0)),
            pl.BlockSpec((1, 1, D), lambda b, h: (h, 0, 0)),
            pl.BlockSpec((1, 1, D), lambda b, h: (h, 0, 0)),
            pl.BlockSpec((1, 1, D), lambda b, h: (h, 0, 0)),
        ],
        out_specs=[pl.BlockSpec((1, 1, N, D), lambda b, h: (b, h, 0, 0))] * 3,
        out_shape=[qkv_shape] * 3,
    )(x, Wq, Wk, Wv, bq3, bk3, bv3)

    attn = pl.pallas_call(
        _attn_kernel,
        grid=(B, H, N // TQ),
        in_specs=[
            pl.BlockSpec((1, 1, TQ, D), lambda b, h, t: (b, h, t, 0)),
            pl.BlockSpec((1, 1, N, D), lambda b, h, t: (b, h, 0, 0)),
            pl.BlockSpec((1, 1, N, D), lambda b, h, t: (b, h, 0, 0)),
            pl.BlockSpec((3, 1, TQ, 1), lambda b, h, t: (0, b, t, 0)),
            pl.BlockSpec((3, 1, 1, N), lambda b, h, t: (0, b, 0, 0)),
        ],
        out_specs=pl.BlockSpec((1, TQ, D), lambda b, h, t: (b, t, h)),
        out_shape=jax.ShapeDtypeStruct((B, N, E), f32),
    )(q, k, v, posc, posr)

    a2 = attn.reshape(B * N, E)
    out = pl.pallas_call(
        _oproj_kernel,
        grid=(B * N // TM,),
        in_specs=[
            pl.BlockSpec((TM, E), lambda t: (t, 0)),
            pl.BlockSpec((E, E), lambda t: (0, 0)),
            pl.BlockSpec((1, E), lambda t: (0, 0)),
        ],
        out_specs=pl.BlockSpec((TM, E), lambda t: (t, 0)),
        out_shape=jax.ShapeDtypeStruct((B * N, E), f32),
    )(a2, Wo, bo.reshape(1, E))
    return out.reshape(B, N, E)


# speedup vs baseline: 12.0624x; 1.1914x over previous
import math

import jax
import jax.numpy as jnp
from jax import lax
from jax.experimental import pallas as pl
from jax.experimental.pallas import tpu as pltpu
from jax.experimental.pallas import tpu_sc as plsc

_CSIZES = (512, 128, 32)
_NUM_HEADS = 16
_NBMAX = 128


def _make_sc_rank(bsz):
    del bsz
    def _sc_rank_body(wb_hbm, pos_hbm, wb_v, pos_v, run_s):
        cid = lax.axis_index("c")
        sid = lax.axis_index("s")
        wid = sid * 2 + cid
        nrows = wb_hbm.shape[0]
        n = wb_hbm.shape[1]

        @pl.when(wid < nrows)
        def _():
            pltpu.sync_copy(wb_hbm.at[wid], wb_v)
            io = lax.iota(jnp.int32, 16)

            def zero_body(c, carry):
                run_s[c] = jnp.int32(0)
                return carry

            lax.fori_loop(0, _NBMAX, zero_body, jnp.int32(0))

            def hist_body(i, carry):
                seg = wb_v[pl.ds(i * 16, 16)]
                for l in range(16):
                    b = seg[l]
                    run_s[b] = run_s[b] + 1
                return carry

            lax.fori_loop(0, n // 16, hist_body, jnp.int32(0))

            def pref_body(c, acc):
                h = run_s[c]
                run_s[c] = acc
                return acc + h

            lax.fori_loop(0, _NBMAX, pref_body, jnp.int32(0))

            def pos_body(i, carry):
                seg = wb_v[pl.ds(i * 16, 16)]
                acc = jnp.zeros((16,), jnp.int32)
                for l in range(16):
                    b = seg[l]
                    r = run_s[b]
                    run_s[b] = r + jnp.int32(1)
                    acc = jnp.where(io == l, r, acc)
                pos_v[pl.ds(i * 16, 16)] = acc
                return carry

            lax.fori_loop(0, n // 16, pos_body, jnp.int32(0))
            pltpu.sync_copy(pos_v, pos_hbm.at[wid])

    return _sc_rank_body


def _qkv_kernel(x_ref, wq_ref, wk_ref, wv_ref, bq_ref, bk_ref, bv_ref,
                q_ref, k_ref, v_ref):
    xb = x_ref[0]
    dn = (((1,), (1,)), ((), ()))
    q_ref[0, 0] = (jax.lax.dot_general(xb, wq_ref[...], dn,
                                       preferred_element_type=jnp.float32)
                   + bq_ref[0])
    k_ref[0, 0] = (jax.lax.dot_general(xb, wk_ref[...], dn,
                                       preferred_element_type=jnp.float32)
                   + bk_ref[0])
    v_ref[0, 0] = (jax.lax.dot_general(xb, wv_ref[...], dn,
                                       preferred_element_type=jnp.float32)
                   + bv_ref[0])


def _chunk_bases(n):
    bases = []
    b = 0
    for cs in _CSIZES:
        bases.append(b)
        b += n // cs
    return bases


def _mmat_kernel(posc_ref, m_ref):
    n = posc_ref.shape[2]
    bases = _chunk_bases(n)
    ci = jax.lax.broadcasted_iota(jnp.int32, (n, 128), 1)
    acc = jnp.zeros((n, 128), jnp.float32)
    for lvl, cs in enumerate(_CSIZES):
        shift = cs.bit_length() - 1
        colid = (posc_ref[lvl, 0] >> shift) + bases[lvl]
        acc = acc + jnp.where(ci == colid, 1.0, 0.0)
    m_ref[0] = acc


def _attn_kernel(q_ref, k_ref, v_ref, posq_ref, mm_ref, o_ref):
    q = q_ref[0, 0]
    k = k_ref[0, 0]
    v = v_ref[0, 0]
    tq, d = q.shape
    n = k.shape[0]
    bases = _chunk_bases(n)
    qs = q * (1.0 / math.sqrt(d))
    s = jax.lax.dot_general(qs, k, (((1,), (1,)), ((), ())),
                            preferred_element_type=jnp.float32)
    m = jnp.maximum(jnp.max(s, axis=1, keepdims=True), 0.0)
    e = jnp.exp(s - m)
    expm = jnp.exp(-m)
    mm = mm_ref[0]
    csum = jax.lax.dot_general(e, mm, (((1,), (0,)), ((), ())),
                               preferred_element_type=jnp.float32)
    ci = jax.lax.broadcasted_iota(jnp.int32, (tq, 128), 1)
    a = jnp.zeros((tq, 128), jnp.float32)
    for lvl, cs in enumerate(_CSIZES):
        shift = cs.bit_length() - 1
        cq = posq_ref[lvl, 0] >> shift
        cqb = cq + bases[lvl]
        m2 = (ci == cqb) | ((ci == cqb - 1) & (cq > 0))
        den = jnp.sum(jnp.where(m2, csum, 0.0), axis=1, keepdims=True)
        den = den + jnp.where(cq == 0, jnp.float32(cs) * expm, 0.0)
        r = 1.0 / jnp.maximum(den, jnp.float32(1e-37))
        a = a + jnp.where(m2, r, 0.0)
    w = jax.lax.dot_general(a, mm, (((1,), (1,)), ((), ())),
                            preferred_element_type=jnp.float32)
    p = e * w
    o = jax.lax.dot_general(p, v, (((1,), (0,)), ((), ())),
                            preferred_element_type=jnp.float32)
    o_ref[0] = o * (1.0 / 3.0)


def _oproj_kernel(a_ref, wo_ref, bo_ref, o_ref):
    o_ref[...] = (jax.lax.dot_general(a_ref[...], wo_ref[...],
                                      (((1,), (1,)), ((), ())),
                                      preferred_element_type=jnp.float32)
                  + bo_ref[...])


def kernel(x, wb_coarse, wb_mid, wb_fine, Wq, bq, Wk, bk, Wv, bv, Wo, bo):
    B, N, E = x.shape
    H = _NUM_HEADS
    D = E // H
    TQ = 512
    TM = 512
    f32 = jnp.float32

    wb_all = jnp.stack([wb_coarse, wb_mid, wb_fine]).astype(jnp.int32)
    wb6 = wb_all.reshape(3 * B, N)
    rank_call = pl.kernel(
        _make_sc_rank(B),
        out_type=jax.ShapeDtypeStruct((3 * B, N), jnp.int32),
        mesh=plsc.VectorSubcoreMesh(core_axis_name="c", subcore_axis_name="s"),
        scratch_types=[
            pltpu.VMEM((N,), jnp.int32),
            pltpu.VMEM((N,), jnp.int32),
            pltpu.SMEM((_NBMAX,), jnp.int32),
        ],
    )
    posc = rank_call(wb6).reshape(3, B, N, 1)

    mmat = pl.pallas_call(
        _mmat_kernel,
        grid=(B,),
        in_specs=[pl.BlockSpec((3, 1, N, 1), lambda b: (0, b, 0, 0))],
        out_specs=pl.BlockSpec((1, N, 128), lambda b: (b, 0, 0)),
        out_shape=jax.ShapeDtypeStruct((B, N, 128), f32),
    )(posc)

    bq3 = bq.reshape(H, 1, D)
    bk3 = bk.reshape(H, 1, D)
    bv3 = bv.reshape(H, 1, D)
    qkv_shape = jax.ShapeDtypeStruct((B, H, N, D), f32)
    q, k, v = pl.pallas_call(
        _qkv_kernel,
        grid=(B, H),
        in_specs=[
            pl.BlockSpec((1, N, E), lambda b, h: (b, 0, 0)),
            pl.BlockSpec((D, E), lambda b, h: (h, 0)),
            pl.BlockSpec((D, E), lambda b, h: (h, 0)),
            pl.BlockSpec((D, E), lambda b, h: (h, 0)),
            pl.BlockSpec((1, 1, D), lambda b, h: (h, 0, 0)),
            pl.BlockSpec((1, 1, D), lambda b, h: (h, 0, 0)),
            pl.BlockSpec((1, 1, D), lambda b, h: (h, 0, 0)),
        ],
        out_specs=[pl.BlockSpec((1, 1, N, D), lambda b, h: (b, h, 0, 0))] * 3,
        out_shape=[qkv_shape] * 3,
    )(x, Wq, Wk, Wv, bq3, bk3, bv3)

    attn = pl.pallas_call(
        _attn_kernel,
        grid=(B, H, N // TQ),
        in_specs=[
            pl.BlockSpec((1, 1, TQ, D), lambda b, h, t: (b, h, t, 0)),
            pl.BlockSpec((1, 1, N, D), lambda b, h, t: (b, h, 0, 0)),
            pl.BlockSpec((1, 1, N, D), lambda b, h, t: (b, h, 0, 0)),
            pl.BlockSpec((3, 1, TQ, 1), lambda b, h, t: (0, b, t, 0)),
            pl.BlockSpec((1, N, 128), lambda b, h, t: (b, 0, 0)),
        ],
        out_specs=pl.BlockSpec((1, TQ, D), lambda b, h, t: (b, t, h)),
        out_shape=jax.ShapeDtypeStruct((B, N, E), f32),
    )(q, k, v, posc, mmat)

    a2 = attn.reshape(B * N, E)
    out = pl.pallas_call(
        _oproj_kernel,
        grid=(B * N // TM,),
        in_specs=[
            pl.BlockSpec((TM, E), lambda t: (t, 0)),
            pl.BlockSpec((E, E), lambda t: (0, 0)),
            pl.BlockSpec((1, E), lambda t: (0, 0)),
        ],
        out_specs=pl.BlockSpec((TM, E), lambda t: (t, 0)),
        out_shape=jax.ShapeDtypeStruct((B * N, E), f32),
    )(a2, Wo, bo.reshape(1, E))
    return out.reshape(B, N, E)


# speedup vs baseline: 12.6190x; 1.0461x over previous
import math

import jax
import jax.numpy as jnp
from jax import lax
from jax.experimental import pallas as pl
from jax.experimental.pallas import tpu as pltpu
from jax.experimental.pallas import tpu_sc as plsc

_CSIZES = (512, 128, 32)
_NUM_HEADS = 16
_NBMAX = 128


def _make_sc_rank(bsz):
    del bsz
    def _sc_rank_body(wb_hbm, pos_hbm, wb_v, pos_v, run_s):
        cid = lax.axis_index("c")
        sid = lax.axis_index("s")
        wid = sid * 2 + cid
        nrows = wb_hbm.shape[0]
        n = wb_hbm.shape[1]

        @pl.when(wid < nrows)
        def _():
            pltpu.sync_copy(wb_hbm.at[wid], wb_v)
            io = lax.iota(jnp.int32, 16)

            def zero_body(c, carry):
                run_s[c] = jnp.int32(0)
                return carry

            lax.fori_loop(0, _NBMAX, zero_body, jnp.int32(0))

            def hist_body(i, carry):
                seg = wb_v[pl.ds(i * 16, 16)]
                for l in range(16):
                    b = seg[l]
                    run_s[b] = run_s[b] + 1
                return carry

            lax.fori_loop(0, n // 16, hist_body, jnp.int32(0))

            def pref_body(c, acc):
                h = run_s[c]
                run_s[c] = acc
                return acc + h

            lax.fori_loop(0, _NBMAX, pref_body, jnp.int32(0))

            def pos_body(i, carry):
                seg = wb_v[pl.ds(i * 16, 16)]
                acc = jnp.zeros((16,), jnp.int32)
                for l in range(16):
                    b = seg[l]
                    r = run_s[b]
                    run_s[b] = r + jnp.int32(1)
                    acc = jnp.where(io == l, r, acc)
                pos_v[pl.ds(i * 16, 16)] = acc
                return carry

            lax.fori_loop(0, n // 16, pos_body, jnp.int32(0))
            pltpu.sync_copy(pos_v, pos_hbm.at[wid])

    return _sc_rank_body


def _qkv_kernel(x_ref, wq_ref, wk_ref, wv_ref, bq_ref, bk_ref, bv_ref,
                q_ref, k_ref, v_ref):
    xb = x_ref[0]
    dn = (((1,), (1,)), ((), ()))
    q_ref[0, 0] = (jax.lax.dot_general(xb, wq_ref[...], dn,
                                       preferred_element_type=jnp.float32)
                   + bq_ref[0])
    k_ref[0, 0] = (jax.lax.dot_general(xb, wk_ref[...], dn,
                                       preferred_element_type=jnp.float32)
                   + bk_ref[0])
    v_ref[0, 0] = (jax.lax.dot_general(xb, wv_ref[...], dn,
                                       preferred_element_type=jnp.float32)
                   + bv_ref[0])


def _chunk_bases(n):
    bases = []
    b = 0
    for cs in _CSIZES:
        bases.append(b)
        b += n // cs
    return bases


def _mmat_kernel(posc_ref, m_ref):
    n = posc_ref.shape[2]
    bases = _chunk_bases(n)
    ci = jax.lax.broadcasted_iota(jnp.int32, (n, 128), 1)
    acc = jnp.zeros((n, 128), jnp.float32)
    for lvl, cs in enumerate(_CSIZES):
        shift = cs.bit_length() - 1
        colid = (posc_ref[lvl, 0] >> shift) + bases[lvl]
        acc = acc + jnp.where(ci == colid, 1.0, 0.0)
    m_ref[0] = acc


def _make_fused_kernel(tq):
    def _fused_kernel(x_ref, wq_ref, wk_ref, wv_ref, bq_ref, bk_ref, bv_ref,
                      posq_ref, mm_ref, o_ref):
        xb = x_ref[0]
        dn = (((1,), (1,)), ((), ()))
        q = (jax.lax.dot_general(xb, wq_ref[...], dn,
                                 preferred_element_type=jnp.float32)
             + bq_ref[0])
        k = (jax.lax.dot_general(xb, wk_ref[...], dn,
                                 preferred_element_type=jnp.float32)
             + bk_ref[0])
        v = (jax.lax.dot_general(xb, wv_ref[...], dn,
                                 preferred_element_type=jnp.float32)
             + bv_ref[0])
        n, d = q.shape
        bases = _chunk_bases(n)
        qs = q * (1.0 / math.sqrt(d))
        mm = mm_ref[0]
        ci = jax.lax.broadcasted_iota(jnp.int32, (tq, 128), 1)
        for t in range(n // tq):
            qt = qs[t * tq:(t + 1) * tq]
            s = jax.lax.dot_general(qt, k, (((1,), (1,)), ((), ())),
                                    preferred_element_type=jnp.float32)
            m = jnp.maximum(jnp.max(s, axis=1, keepdims=True), 0.0)
            e = jnp.exp(s - m)
            expm = jnp.exp(-m)
            csum = jax.lax.dot_general(e, mm, (((1,), (0,)), ((), ())),
                                       preferred_element_type=jnp.float32)
            a = jnp.zeros((tq, 128), jnp.float32)
            for lvl, cs in enumerate(_CSIZES):
                shift = cs.bit_length() - 1
                cq = posq_ref[lvl, 0, t * tq:(t + 1) * tq, :]
                cq = cq >> shift
                cqb = cq + bases[lvl]
                m2 = (ci == cqb) | ((ci == cqb - 1) & (cq > 0))
                den = jnp.sum(jnp.where(m2, csum, 0.0), axis=1,
                              keepdims=True)
                den = den + jnp.where(cq == 0, jnp.float32(cs) * expm, 0.0)
                r = 1.0 / jnp.maximum(den, jnp.float32(1e-37))
                a = a + jnp.where(m2, r, 0.0)
            w = jax.lax.dot_general(a, mm, (((1,), (1,)), ((), ())),
                                    preferred_element_type=jnp.float32)
            p = e * w
            o = jax.lax.dot_general(p, v, (((1,), (0,)), ((), ())),
                                    preferred_element_type=jnp.float32)
            o_ref[0, t * tq:(t + 1) * tq, :] = o * (1.0 / 3.0)

    return _fused_kernel


def _oproj_kernel(a_ref, wo_ref, bo_ref, o_ref):
    o_ref[...] = (jax.lax.dot_general(a_ref[...], wo_ref[...],
                                      (((1,), (1,)), ((), ())),
                                      preferred_element_type=jnp.float32)
                  + bo_ref[...])


def kernel(x, wb_coarse, wb_mid, wb_fine, Wq, bq, Wk, bk, Wv, bv, Wo, bo):
    B, N, E = x.shape
    H = _NUM_HEADS
    D = E // H
    TQ = 256
    TM = 512
    f32 = jnp.float32

    wb_all = jnp.stack([wb_coarse, wb_mid, wb_fine]).astype(jnp.int32)
    wb6 = wb_all.reshape(3 * B, N)
    rank_call = pl.kernel(
        _make_sc_rank(B),
        out_type=jax.ShapeDtypeStruct((3 * B, N), jnp.int32),
        mesh=plsc.VectorSubcoreMesh(core_axis_name="c", subcore_axis_name="s"),
        scratch_types=[
            pltpu.VMEM((N,), jnp.int32),
            pltpu.VMEM((N,), jnp.int32),
            pltpu.SMEM((_NBMAX,), jnp.int32),
        ],
    )
    posc = rank_call(wb6).reshape(3, B, N, 1)

    mmat = pl.pallas_call(
        _mmat_kernel,
        grid=(B,),
        in_specs=[pl.BlockSpec((3, 1, N, 1), lambda b: (0, b, 0, 0))],
        out_specs=pl.BlockSpec((1, N, 128), lambda b: (b, 0, 0)),
        out_shape=jax.ShapeDtypeStruct((B, N, 128), f32),
    )(posc)

    bq3 = bq.reshape(H, 1, D)
    bk3 = bk.reshape(H, 1, D)
    bv3 = bv.reshape(H, 1, D)
    attn = pl.pallas_call(
        _make_fused_kernel(TQ),
        grid=(B, H),
        in_specs=[
            pl.BlockSpec((1, N, E), lambda b, h: (b, 0, 0)),
            pl.BlockSpec((D, E), lambda b, h: (h, 0)),
            pl.BlockSpec((D, E), lambda b, h: (h, 0)),
            pl.BlockSpec((D, E), lambda b, h: (h, 0)),
            pl.BlockSpec((1, 1, D), lambda b, h: (h, 0, 0)),
            pl.BlockSpec((1, 1, D), lambda b, h: (h, 0, 0)),
            pl.BlockSpec((1, 1, D), lambda b, h: (h, 0, 0)),
            pl.BlockSpec((3, 1, N, 1), lambda b, h: (0, b, 0, 0)),
            pl.BlockSpec((1, N, 128), lambda b, h: (b, 0, 0)),
        ],
        out_specs=pl.BlockSpec((1, N, D), lambda b, h: (b, 0, h)),
        out_shape=jax.ShapeDtypeStruct((B, N, E), f32),
    )(x, Wq, Wk, Wv, bq3, bk3, bv3, posc, mmat)

    a2 = attn.reshape(B * N, E)
    out = pl.pallas_call(
        _oproj_kernel,
        grid=(B * N // TM,),
        in_specs=[
            pl.BlockSpec((TM, E), lambda t: (t, 0)),
            pl.BlockSpec((E, E), lambda t: (0, 0)),
            pl.BlockSpec((1, E), lambda t: (0, 0)),
        ],
        out_specs=pl.BlockSpec((TM, E), lambda t: (t, 0)),
        out_shape=jax.ShapeDtypeStruct((B * N, E), f32),
    )(a2, Wo, bo.reshape(1, E))
    return out.reshape(B, N, E)
